# baseline (speedup 1.0000x reference)
import functools

import jax
import jax.numpy as jnp
from jax import lax
from jax.experimental import pallas as pl
from jax.experimental.pallas import tpu as pltpu
from jax.experimental.pallas import tpu_sc as plsc

VOCAB = 100000
EMB = 300
NOUT = 2
BATCH = 4096
HIST = 50
LANES = 16
NC, NS = 2, 16
NW = NC * NS
B_PER_W = BATCH // NW
NGRP = B_PER_W // LANES
K_BLK = 64
K_STEPS = -(-EMB // K_BLK)
K_PAD = K_BLK * K_STEPS


def _matmul_body(wt_ref, w_ref, opk_ref, a0_ref, a1_ref):
    part = lax.dot_general(wt_ref[...], w_ref[...],
                           (((0,), (0,)), ((), ())),
                           preferred_element_type=jnp.float32)

    @pl.when(pl.program_id(0) == 0)
    def _():
        a0_ref[...] = part[0]
        a1_ref[...] = part[1]

    @pl.when(pl.program_id(0) > 0)
    def _():
        a0_ref[...] = a0_ref[...] + part[0]
        a1_ref[...] = a1_ref[...] + part[1]

    @pl.when(pl.program_id(0) == K_STEPS - 1)
    def _():
        b0 = lax.bitcast_convert_type(
            a0_ref[...].astype(jnp.bfloat16), jnp.uint16).astype(jnp.uint32)
        b1 = lax.bitcast_convert_type(
            a1_ref[...].astype(jnp.bfloat16), jnp.uint16).astype(jnp.uint32)
        opk_ref[...] = lax.bitcast_convert_type(
            b0 | (b1 << jnp.uint32(16)), jnp.int32)


def _project(wT, wtp):
    return pl.pallas_call(
        _matmul_body,
        grid=(K_STEPS,),
        in_specs=[
            pl.BlockSpec((K_BLK, 8), lambda i: (i, 0)),
            pl.BlockSpec((K_BLK, VOCAB), lambda i: (i, 0)),
        ],
        out_specs=pl.BlockSpec((VOCAB,), lambda i: (0,)),
        out_shape=jax.ShapeDtypeStruct((VOCAB,), jnp.int32),
        scratch_shapes=[pltpu.VMEM((VOCAB,), jnp.float32),
                        pltpu.VMEM((VOCAB,), jnp.float32)],
        compiler_params=pltpu.CompilerParams(vmem_limit_bytes=56 * 2**20),
    )(wtp, wT)


def _pool_body(pk_hbm, xt_hbm, out_hbm, tbl_v, xt_v, o0_v, o1_v, scale_v,
               tbl_sem, xt_sem):
    wid = lax.axis_index("s") * NC + lax.axis_index("c")
    r0 = wid * B_PER_W

    xt_copy = pltpu.async_copy(xt_hbm.at[:, pl.ds(r0, B_PER_W)], xt_v, xt_sem)
    tbl_copy = pltpu.async_copy(pk_hbm, tbl_v, tbl_sem)
    scale_v[...] = jnp.full((LANES,), 1.0 / HIST, jnp.float32)
    xt_copy.wait()
    tbl_copy.wait()
    himask = jnp.full((LANES,), -65536, jnp.int32)

    @pl.loop(0, NGRP)
    def _grp(g):
        acc0 = jnp.zeros((LANES,), jnp.float32)
        acc1 = jnp.zeros((LANES,), jnp.float32)
        for l in range(HIST):
            idx = xt_v[l, pl.ds(g * LANES, LANES)]
            bits = plsc.load_gather(tbl_v, [idx])
            acc0 = acc0 + plsc.bitcast(bits << 16, jnp.float32)
            acc1 = acc1 + plsc.bitcast(bits & himask, jnp.float32)
        o0_v[pl.ds(g * LANES, LANES)] = acc0 * scale_v[...]
        o1_v[pl.ds(g * LANES, LANES)] = acc1 * scale_v[...]

    pltpu.sync_copy(o0_v, out_hbm.at[0, pl.ds(r0, B_PER_W)])
    pltpu.sync_copy(o1_v, out_hbm.at[1, pl.ds(r0, B_PER_W)])


@functools.cache
def _pool():
    return pl.kernel(
        _pool_body,
        out_type=jax.ShapeDtypeStruct((NOUT, BATCH), jnp.float32),
        mesh=plsc.VectorSubcoreMesh(core_axis_name="c", subcore_axis_name="s",
                                    num_cores=NC, num_subcores=NS),
        compiler_params=pltpu.CompilerParams(use_tc_tiling_on_sc=False,
                                             needs_layout_passes=False),
        scratch_types=[
            pltpu.VMEM((VOCAB,), jnp.int32),
            pltpu.VMEM((HIST, B_PER_W), jnp.int32),
            pltpu.VMEM((B_PER_W,), jnp.float32),
            pltpu.VMEM((B_PER_W,), jnp.float32),
            pltpu.VMEM((LANES,), jnp.float32),
            pltpu.SemaphoreType.DMA,
            pltpu.SemaphoreType.DMA,
        ],
    )


def kernel(x, weight, W_out):
    wtp = jnp.zeros((K_PAD, 8), jnp.float32).at[:EMB, :NOUT].set(W_out.T)
    pk = _project(weight.T, wtp)
    xt = x.astype(jnp.int32).T
    pooled = _pool()(pk, xt)
    return pooled.T
